# baseline (speedup 1.0000x reference)
import functools

import jax
import jax.numpy as jnp
from jax import lax
from jax.experimental import pallas as pl
from jax.experimental.pallas import tpu as pltpu
from jax.experimental.pallas import tpu_sc as plsc

_BATCH = 16384
_HIST = 50
_DIM = 32
_N = _BATCH * _HIST
_NC = 2
_NS = 16
_NW = _NC * _NS
_BB = _BATCH // _NW
_L = 16


@functools.partial(
    pl.kernel,
    out_type=jax.ShapeDtypeStruct((_HIST, _DIM, _BATCH), jnp.float32),
    mesh=plsc.VectorSubcoreMesh(core_axis_name="c", subcore_axis_name="s"),
    scratch_types=[
        pltpu.VMEM((_BB * _HIST,), jnp.int32),
        pltpu.VMEM((_HIST, _BB), jnp.int32),
        pltpu.VMEM((_BB, _DIM), jnp.float32),
        pltpu.VMEM((_BB, _DIM), jnp.float32),
        pltpu.VMEM((_DIM, _BB), jnp.float32),
        pltpu.VMEM((_DIM, _BB), jnp.float32),
        pltpu.SemaphoreType.DMA,
        pltpu.SemaphoreType.DMA,
        pltpu.SemaphoreType.DMA,
        pltpu.SemaphoreType.DMA,
    ],
    compiler_params=pltpu.CompilerParams(use_tc_tiling_on_sc=False, needs_layout_passes=False),
)
def _gather_kernel(idx_hbm, table_hbm, out_hbm, idx_v, idxT, rows0, rows1,
                   t0, t1, g0, g1, s0, s1):
    wid = lax.axis_index("s") * _NC + lax.axis_index("c")
    b0 = wid * _BB
    rows = (rows0, rows1)
    tbuf = (t0, t1)
    gsem = (g0, g1)
    ssem = (s0, s1)
    iota = lax.iota(jnp.int32, _L)

    pltpu.sync_copy(idx_hbm.at[pl.ds(b0 * _HIST, _BB * _HIST)], idx_v)

    def idx_t(h, _):
        def blk(k, _):
            src = (k * _L + iota) * _HIST + h
            idxT[h, pl.ds(k * _L, _L)] = plsc.load_gather(idx_v, [src])
            return ()
        lax.fori_loop(0, _BB // _L, blk, ())
        return ()
    lax.fori_loop(0, _HIST, idx_t, ())

    def gather(h, b):
        return pltpu.async_copy(table_hbm.at[idxT.at[h]], rows[b], gsem[b])

    def gather_wait(h, b):
        pltpu.make_async_copy(
            table_hbm.at[idxT.at[h]], rows[b], gsem[b]).wait()

    def store(h, b):
        return pltpu.async_copy(
            tbuf[b], out_hbm.at[h, :, pl.ds(b0, _BB)], ssem[b])

    def store_wait(h, b):
        pltpu.make_async_copy(
            tbuf[b], out_hbm.at[h, :, pl.ds(b0, _BB)], ssem[b]).wait()

    gather(0, 0)

    def pair(g, _):
        for b in (0, 1):
            h = 2 * g + b
            gather_wait(h, b)
            @pl.when(h + 1 < _HIST)
            def _():
                gather(h + 1, 1 - b)
            @pl.when(h >= 2)
            def _():
                store_wait(h - 2, b)

            def tr(k, _, b=b):
                src_row = k * _L + iota
                for j in range(_DIM):
                    col = jnp.full((_L,), j, jnp.int32)
                    tbuf[b][j, pl.ds(k * _L, _L)] = plsc.load_gather(
                        rows[b], [src_row, col])
                return ()
            lax.fori_loop(0, _BB // _L, tr, ())

            store(h, b)
        return ()

    lax.fori_loop(0, _HIST // 2, pair, ())
    store_wait(_HIST - 2, 0)
    store_wait(_HIST - 1, 1)


def kernel(batch, table):
    idx = batch.reshape(_N).astype(jnp.int32)
    out = _gather_kernel(idx, table)
    return out.transpose(2, 0, 1)


# speedup vs baseline: 1.2369x; 1.2369x over previous
import functools

import jax
import jax.numpy as jnp
from jax import lax
from jax.experimental import pallas as pl
from jax.experimental.pallas import tpu as pltpu
from jax.experimental.pallas import tpu_sc as plsc

_BATCH = 16384
_HIST = 50
_DIM = 32
_N = _BATCH * _HIST
_NC = 2
_NS = 16
_NW = _NC * _NS
_BB = _BATCH // _NW
_L = 16


@functools.partial(
    pl.kernel,
    out_type=jax.ShapeDtypeStruct((_HIST, _DIM, _BATCH), jnp.float32),
    mesh=plsc.VectorSubcoreMesh(core_axis_name="c", subcore_axis_name="s"),
    scratch_types=[
        pltpu.VMEM((_BB * _HIST,), jnp.int32),
        pltpu.VMEM((_HIST, _BB), jnp.int32),
        pltpu.VMEM((_BB, _DIM), jnp.float32),
        pltpu.VMEM((_BB, _DIM), jnp.float32),
        pltpu.VMEM((_DIM, _BB), jnp.float32),
        pltpu.VMEM((_DIM, _BB), jnp.float32),
        pltpu.SemaphoreType.DMA,
        pltpu.SemaphoreType.DMA,
        pltpu.SemaphoreType.DMA,
        pltpu.SemaphoreType.DMA,
    ],
    compiler_params=pltpu.CompilerParams(use_tc_tiling_on_sc=False, needs_layout_passes=False),
)
def _gather_kernel(idx_hbm, table_hbm, out_hbm, idx_v, idxT, rows0, rows1,
                   t0, t1, g0, g1, s0, s1):
    wid = lax.axis_index("s") * _NC + lax.axis_index("c")
    b0 = wid * _BB
    rows = (rows0, rows1)
    tbuf = (t0, t1)
    gsem = (g0, g1)
    ssem = (s0, s1)
    iota = lax.iota(jnp.int32, _L)
    cols = tuple(jnp.full((_L,), j, jnp.int32) for j in range(_DIM))

    pltpu.sync_copy(idx_hbm.at[pl.ds(b0 * _HIST, _BB * _HIST)], idx_v)

    def idx_t(h, _):
        @plsc.parallel_loop(0, _BB // _L, unroll=2)
        def _(k):
            src = (k * _L + iota) * _HIST + h
            idxT[h, pl.ds(k * _L, _L)] = plsc.load_gather(idx_v, [src])
        return ()
    lax.fori_loop(0, _HIST, idx_t, ())

    def gather(h, b):
        return pltpu.async_copy(table_hbm.at[idxT.at[h]], rows[b], gsem[b])

    def gather_wait(h, b):
        pltpu.make_async_copy(
            table_hbm.at[idxT.at[h]], rows[b], gsem[b]).wait()

    def store(h, b):
        return pltpu.async_copy(
            tbuf[b], out_hbm.at[h, :, pl.ds(b0, _BB)], ssem[b])

    def store_wait(h, b):
        pltpu.make_async_copy(
            tbuf[b], out_hbm.at[h, :, pl.ds(b0, _BB)], ssem[b]).wait()

    gather(0, 0)

    def pair(g, _):
        for b in (0, 1):
            h = 2 * g + b
            gather_wait(h, b)
            @pl.when(h + 1 < _HIST)
            def _():
                gather(h + 1, 1 - b)
            @pl.when(h >= 2)
            def _():
                store_wait(h - 2, b)

            @plsc.parallel_loop(0, _BB // _L, unroll=2)
            def _(k, b=b):
                src_row = k * _L + iota
                for j in range(_DIM):
                    tbuf[b][j, pl.ds(k * _L, _L)] = plsc.load_gather(
                        rows[b], [src_row, cols[j]])

            store(h, b)
        return ()

    lax.fori_loop(0, _HIST // 2, pair, ())
    store_wait(_HIST - 2, 0)
    store_wait(_HIST - 1, 1)


def kernel(batch, table):
    idx = batch.reshape(_N).astype(jnp.int32)
    out = _gather_kernel(idx, table)
    return out.transpose(2, 0, 1)


# speedup vs baseline: 1.7569x; 1.4204x over previous
import functools

import jax
import jax.numpy as jnp
from jax import lax
from jax.experimental import pallas as pl
from jax.experimental.pallas import tpu as pltpu
from jax.experimental.pallas import tpu_sc as plsc

_BATCH = 16384
_HIST = 50
_DIM = 32
_N = _BATCH * _HIST
_NC = 2
_NS = 16
_NW = _NC * _NS
_BB = _BATCH // _NW
_L = 16


@functools.partial(
    pl.kernel,
    out_type=jax.ShapeDtypeStruct((_HIST, _DIM, _BATCH), jnp.float32),
    mesh=plsc.VectorSubcoreMesh(core_axis_name="c", subcore_axis_name="s"),
    scratch_types=[
        pltpu.VMEM((_BB * _HIST,), jnp.int32),
        pltpu.VMEM((_HIST, _BB), jnp.int32),
        pltpu.VMEM((_BB, _DIM), jnp.float32),
        pltpu.VMEM((_BB, _DIM), jnp.float32),
        pltpu.VMEM((_DIM, _BB + 17), jnp.float32),
        pltpu.VMEM((_DIM, _BB + 17), jnp.float32),
        pltpu.SemaphoreType.DMA,
        pltpu.SemaphoreType.DMA,
        pltpu.SemaphoreType.DMA,
        pltpu.SemaphoreType.DMA,
    ],
    compiler_params=pltpu.CompilerParams(use_tc_tiling_on_sc=False, needs_layout_passes=False),
)
def _gather_kernel(idx_hbm, table_hbm, out_hbm, idx_v, idxT, rows0, rows1,
                   t0, t1, g0, g1, s0, s1):
    wid = lax.axis_index("s") * _NC + lax.axis_index("c")
    b0 = wid * _BB
    rows = (rows0, rows1)
    tbuf = (t0, t1)
    gsem = (g0, g1)
    ssem = (s0, s1)
    iota = lax.iota(jnp.int32, _L)
    cols = tuple(jnp.full((_L,), j, jnp.int32) for j in range(_DIM))

    pltpu.sync_copy(idx_hbm.at[pl.ds(b0 * _HIST, _BB * _HIST)], idx_v)

    def idx_t(h, _):
        @plsc.parallel_loop(0, _BB // _L, unroll=2)
        def _(k):
            src = (k * _L + iota) * _HIST + h
            idxT[h, pl.ds(k * _L, _L)] = plsc.load_gather(idx_v, [src])
        return ()
    lax.fori_loop(0, _HIST, idx_t, ())

    def gather(h, b):
        return pltpu.async_copy(table_hbm.at[idxT.at[h]], rows[b], gsem[b])

    def gather_wait(h, b):
        pltpu.make_async_copy(
            table_hbm.at[idxT.at[h]], rows[b], gsem[b]).wait()

    def store(h, b):
        return pltpu.async_copy(
            tbuf[b].at[:, pl.ds(0, _BB)],
            out_hbm.at[h, :, pl.ds(b0, _BB)], ssem[b])

    def store_wait(h, b):
        pltpu.make_async_copy(
            tbuf[b].at[:, pl.ds(0, _BB)],
            out_hbm.at[h, :, pl.ds(b0, _BB)], ssem[b]).wait()

    gather(0, 0)

    def pair(g, _):
        for b in (0, 1):
            h = 2 * g + b
            gather_wait(h, b)
            @pl.when(h + 1 < _HIST)
            def _():
                gather(h + 1, 1 - b)
            @pl.when(h >= 2)
            def _():
                store_wait(h - 2, b)

            @plsc.parallel_loop(0, _BB, unroll=4)
            def _(r, b=b):
                col = jnp.broadcast_to(r, (_L,)).astype(jnp.int32)
                for u in range(_DIM // _L):
                    v = rows[b][r, pl.ds(u * _L, _L)]
                    plsc.store_scatter(tbuf[b], [iota + u * _L, col], v)

            store(h, b)
        return ()

    lax.fori_loop(0, _HIST // 2, pair, ())
    store_wait(_HIST - 2, 0)
    store_wait(_HIST - 1, 1)


def kernel(batch, table):
    idx = batch.reshape(_N).astype(jnp.int32)
    out = _gather_kernel(idx, table)
    return out.transpose(2, 0, 1)
